# baseline (speedup 1.0000x reference)
import functools

import jax
import jax.numpy as jnp
from jax import lax
from jax.experimental import pallas as pl
from jax.experimental.pallas import tpu as pltpu
from jax.experimental.pallas import tpu_sc as plsc

_NC = 2
_NS = 16
_NW = _NC * _NS
_LANES = 16
_BB = 128
_VC = 128


@functools.lru_cache(maxsize=None)
def _build_retile(vocab: int, dim: int):
    n_full = vocab // _VC
    rem = vocab % _VC
    n_iter = -(-n_full // _NW)
    assert dim % _LANES == 0 and rem % 8 == 0

    mesh = plsc.VectorSubcoreMesh(core_axis_name="c", subcore_axis_name="s")

    @functools.partial(
        pl.kernel,
        mesh=mesh,
        out_type=jax.ShapeDtypeStruct((4 * vocab, dim), jnp.float32),
        compiler_params=pltpu.CompilerParams(needs_layout_passes=False,
                                             use_tc_tiling_on_sc=True),
        scratch_types=[
            pltpu.VMEM((dim, _VC), jnp.float32),
            pltpu.VMEM((_VC, dim), jnp.float32),
            pltpu.VMEM((dim, rem or 8), jnp.float32),
            pltpu.VMEM((rem or 8, dim), jnp.float32),
        ],
    )
    def retile_kernel(t0, t1, t2, t3, out_hbm, in_v, out_v, in_r, out_r):
        wid = lax.axis_index("s") * _NC + lax.axis_index("c")
        iota = lax.iota(jnp.int32, _LANES)
        zeros = jnp.full((_LANES,), 0, jnp.int32)
        dconsts = [iota + h * _LANES for h in range(dim // _LANES)]
        tabs = (t0, t1, t2, t3)

        def transpose_block(src, dst, width):
            @plsc.parallel_loop(0, width, unroll=4)
            def _(vcol):
                v_vec = zeros + vcol
                for h in range(dim // _LANES):
                    w = plsc.load_gather(src, [dconsts[h], v_vec])
                    dst[vcol, pl.ds(h * _LANES, _LANES)] = w

        def chunk_body(k, _):
            c = wid + k * _NW

            @pl.when(c < n_full)
            def _():
                for t in range(4):
                    pltpu.sync_copy(tabs[t].at[:, pl.ds(c * _VC, _VC)], in_v)
                    transpose_block(in_v, out_v, _VC)
                    pltpu.sync_copy(
                        out_v, out_hbm.at[pl.ds(t * vocab + c * _VC, _VC), :])
            return 0

        lax.fori_loop(0, n_iter, chunk_body, 0)

        if rem:
            @pl.when(wid == _NW - 1)
            def _():
                for t in range(4):
                    pltpu.sync_copy(tabs[t].at[:, pl.ds(n_full * _VC, rem)],
                                    in_r)
                    transpose_block(in_r, out_r, rem)
                    pltpu.sync_copy(
                        out_r,
                        out_hbm.at[pl.ds(t * vocab + n_full * _VC, rem), :])

    return retile_kernel


@functools.lru_cache(maxsize=None)
def _build_gather(batch: int, seq: int, vocab: int, dim: int):
    assert batch == _NW * _BB and seq % 2 == 0
    n_bchunk = _BB // _LANES

    mesh = plsc.VectorSubcoreMesh(core_axis_name="c", subcore_axis_name="s")

    @functools.partial(
        pl.kernel,
        mesh=mesh,
        out_type=jax.ShapeDtypeStruct((seq, dim, _NW, 4, _BB), jnp.float32),
        compiler_params=pltpu.CompilerParams(needs_layout_passes=False,
                                             use_tc_tiling_on_sc=False),
        scratch_types=[
            pltpu.VMEM((_BB * seq,), jnp.int32),
            pltpu.VMEM((2, 4, _BB), jnp.int32),
            pltpu.VMEM((2, 4 * _BB, dim), jnp.float32),
            pltpu.VMEM((2, dim, 4, _BB), jnp.float32),
            pltpu.SemaphoreType.DMA,
            pltpu.SemaphoreType.DMA,
            pltpu.SemaphoreType.DMA,
            pltpu.SemaphoreType.DMA,
        ],
    )
    def gather_kernel(x_hbm, tab, out_hbm, xt_v, idx_v, rall, o_v,
                      semg0, semg1, semo0, semo1):
        wid = lax.axis_index("s") * _NC + lax.axis_index("c")
        iota = lax.iota(jnp.int32, _LANES)
        semg = (semg0, semg1)
        semo = (semo0, semo1)
        xrows = [(iota + bc * _LANES) * seq for bc in range(n_bchunk)]
        rrows = [[iota + (t * _BB + bc * _LANES) for bc in range(n_bchunk)]
                 for t in range(4)]
        zeros = jnp.full((_LANES,), 0, jnp.int32)

        pltpu.sync_copy(x_hbm.at[pl.ds(wid * (_BB * seq), _BB * seq)], xt_v)

        def start_gather(l, buf):
            l_vec = zeros + l
            for bc in range(n_bchunk):
                v = plsc.load_gather(xt_v, [xrows[bc] + l_vec])
                for t in range(4):
                    idx_v[buf, t, pl.ds(bc * _LANES, _LANES)] = (
                        v + t * vocab)
            for j in range(4):
                pltpu.async_copy(
                    tab.at[idx_v.at[buf, j]],
                    rall.at[buf, pl.ds(j * _BB, _BB), :], semg[buf])

        def wait_gather(buf):
            for j in range(4):
                pltpu.make_async_copy(
                    tab.at[idx_v.at[buf, j]],
                    rall.at[buf, pl.ds(j * _BB, _BB), :], semg[buf]).wait()

        def interleave(buf):
            rbuf = rall.at[buf]

            @plsc.parallel_loop(0, dim, unroll=4)
            def dim_body(d):
                d_vec = zeros + d
                for t in range(4):
                    for bc in range(n_bchunk):
                        v = plsc.load_gather(rbuf, [rrows[t][bc], d_vec])
                        o_v[buf, d, t, pl.ds(bc * _LANES, _LANES)] = v

        def start_out(l, buf):
            pltpu.async_copy(o_v.at[buf], out_hbm.at[l, :, wid], semo[buf])

        def wait_out(l, buf):
            pltpu.make_async_copy(o_v.at[buf], out_hbm.at[l, :, wid],
                                  semo[buf]).wait()

        start_gather(0, 0)

        def seq_body(i2, _):
            for phase in range(2):
                l = i2 * 2 + phase
                wait_gather(phase)

                @pl.when(l + 1 < seq)
                def _():
                    start_gather(l + 1, 1 - phase)

                @pl.when(i2 > 0)
                def _():
                    wait_out(l - 2, phase)

                interleave(phase)
                start_out(l, phase)
            return 0

        lax.fori_loop(0, seq // 2, seq_body, 0)
        wait_out(seq - 2, 0)
        wait_out(seq - 1, 1)

    return gather_kernel


def kernel(x, scalar, vector_i, vector_j, vector_k):
    b, l = x.shape
    vocab, dim = scalar.shape
    x_flat = x.reshape(b * l).astype(jnp.int32)
    tabs_t = [jnp.swapaxes(t, 0, 1)
              for t in (scalar, vector_i, vector_j, vector_k)]
    tab = _build_retile(vocab, dim)(*tabs_t)
    o5 = _build_gather(b, l, vocab, dim)(x_flat, tab)
    return o5.transpose(2, 4, 0, 1, 3).reshape(b, l, dim, 4)


# speedup vs baseline: 2.4867x; 2.4867x over previous
import functools

import jax
import jax.numpy as jnp
from jax import lax
from jax.experimental import pallas as pl
from jax.experimental.pallas import tpu as pltpu
from jax.experimental.pallas import tpu_sc as plsc

_NC = 2
_NS = 16
_NW = _NC * _NS
_LANES = 16
_BB = 128
_VC = 128


@functools.lru_cache(maxsize=None)
def _build_retile(vocab: int, dim: int):
    n_full = vocab // _VC
    rem = vocab % _VC
    n_iter = -(-n_full // _NW)
    n_iter2 = (n_iter + 1) // 2
    assert dim % _LANES == 0 and rem % 8 == 0
    row = 4 * dim

    mesh = plsc.VectorSubcoreMesh(core_axis_name="c", subcore_axis_name="s")

    @functools.partial(
        pl.kernel,
        mesh=mesh,
        out_type=jax.ShapeDtypeStruct((vocab, row), jnp.float32),
        compiler_params=pltpu.CompilerParams(needs_layout_passes=False,
                                             use_tc_tiling_on_sc=True),
        scratch_types=[
            pltpu.VMEM((2, 4, dim, _VC), jnp.float32),
            pltpu.VMEM((2, _VC, row), jnp.float32),
            pltpu.VMEM((4, (rem or 8) * dim), jnp.float32),
            pltpu.VMEM((rem or 8, row), jnp.float32),
            pltpu.SemaphoreType.DMA,
            pltpu.SemaphoreType.DMA,
            pltpu.SemaphoreType.DMA,
            pltpu.SemaphoreType.DMA,
        ],
    )
    def retile_kernel(t0, t1, t2, t3, r0, r1, r2, r3, out_hbm, in_v, out_v,
                      in_r, out_r, semi0, semi1, semu0, semu1):
        wid = lax.axis_index("s") * _NC + lax.axis_index("c")
        iota = lax.iota(jnp.int32, _LANES)
        zeros = jnp.full((_LANES,), 0, jnp.int32)
        dconsts = [iota + h * _LANES for h in range(dim // _LANES)]
        tabs = (t0, t1, t2, t3)
        semi = (semi0, semi1)
        semu = (semu0, semu1)

        def fire_in(c, buf):
            for t in range(4):
                pltpu.async_copy(tabs[t].at[:, pl.ds(c * _VC, _VC)],
                                 in_v.at[buf, t], semi[buf])

        def wait_in(c, buf):
            for t in range(4):
                pltpu.make_async_copy(tabs[t].at[:, pl.ds(c * _VC, _VC)],
                                      in_v.at[buf, t], semi[buf]).wait()

        def out_cp(c, buf):
            return pltpu.make_async_copy(
                out_v.at[buf], out_hbm.at[pl.ds(c * _VC, _VC), :], semu[buf])

        def transpose_block(src, dst, width, col0):
            @plsc.parallel_loop(0, width, unroll=4)
            def _(vcol):
                v_vec = zeros + vcol
                for h in range(dim // _LANES):
                    w = plsc.load_gather(src, [dconsts[h], v_vec])
                    dst[vcol, pl.ds(col0 + h * _LANES, _LANES)] = w

        fire_in(wid, 0)

        def chunk_body(k2, _):
            for p in range(2):
                k = k2 * 2 + p
                c = wid + k * _NW

                @pl.when(c < n_full)
                def _():
                    wait_in(c, p)

                    @pl.when(c + _NW < n_full)
                    def _():
                        fire_in(c + _NW, 1 - p)

                    @pl.when(k >= 2)
                    def _():
                        out_cp(0, p).wait()

                    for t in range(4):
                        transpose_block(in_v.at[p, t], out_v.at[p], _VC,
                                        t * dim)
                    pltpu.async_copy(out_v.at[p],
                                     out_hbm.at[pl.ds(c * _VC, _VC), :],
                                     semu[p])
            return 0

        lax.fori_loop(0, n_iter2, chunk_body, 0)
        out_cp(0, 0).wait()
        out_cp(0, 1).wait()

        if rem:
            rems = (r0, r1, r2, r3)

            @pl.when(wid == _NW - 1)
            def _():
                for t in range(4):
                    pltpu.sync_copy(rems[t], in_r.at[t])

                @plsc.parallel_loop(0, rem, unroll=4)
                def _(v):
                    for t in range(4):
                        for h in range(dim // _LANES):
                            w = in_r[t, pl.ds(v * dim + h * _LANES, _LANES)]
                            out_r[v, pl.ds(t * dim + h * _LANES, _LANES)] = w

                pltpu.sync_copy(out_r,
                                out_hbm.at[pl.ds(n_full * _VC, rem), :])

    return retile_kernel


@functools.lru_cache(maxsize=None)
def _build_gather(batch: int, seq: int, vocab: int, dim: int):
    assert batch == _NW * _BB and seq % 2 == 0
    n_bchunk = _BB // _LANES
    row = 4 * dim

    mesh = plsc.VectorSubcoreMesh(core_axis_name="c", subcore_axis_name="s")

    @functools.partial(
        pl.kernel,
        mesh=mesh,
        out_type=jax.ShapeDtypeStruct((seq, dim, _NW, 4, _BB), jnp.float32),
        compiler_params=pltpu.CompilerParams(needs_layout_passes=False,
                                             use_tc_tiling_on_sc=False),
        scratch_types=[
            pltpu.VMEM((_BB * seq,), jnp.int32),
            pltpu.VMEM((2, _BB), jnp.int32),
            pltpu.VMEM((2, _BB, row), jnp.float32),
            pltpu.VMEM((2, dim, 4, _BB), jnp.float32),
            pltpu.SemaphoreType.DMA,
            pltpu.SemaphoreType.DMA,
            pltpu.SemaphoreType.DMA,
            pltpu.SemaphoreType.DMA,
        ],
    )
    def gather_kernel(x_hbm, tab, out_hbm, xt_v, idx_v, rall, o_v,
                      semg0, semg1, semo0, semo1):
        wid = lax.axis_index("s") * _NC + lax.axis_index("c")
        iota = lax.iota(jnp.int32, _LANES)
        semg = (semg0, semg1)
        semo = (semo0, semo1)
        xrows = [(iota + bc * _LANES) * seq for bc in range(n_bchunk)]
        brows = [iota + bc * _LANES for bc in range(n_bchunk)]
        zeros = jnp.full((_LANES,), 0, jnp.int32)

        pltpu.sync_copy(x_hbm.at[pl.ds(wid * (_BB * seq), _BB * seq)], xt_v)

        def start_gather(l, buf):
            l_vec = zeros + l
            for bc in range(n_bchunk):
                idx_v[buf, pl.ds(bc * _LANES, _LANES)] = plsc.load_gather(
                    xt_v, [xrows[bc] + l_vec])
            pltpu.async_copy(tab.at[idx_v.at[buf]], rall.at[buf], semg[buf])

        def wait_gather(buf):
            pltpu.make_async_copy(tab.at[idx_v.at[buf]], rall.at[buf],
                                  semg[buf]).wait()

        def interleave(buf):
            rbuf = rall.at[buf]

            @plsc.parallel_loop(0, dim, unroll=4)
            def dim_body(d):
                d_vec = zeros + d
                for t in range(4):
                    c_vec = d_vec + t * dim
                    for bc in range(n_bchunk):
                        v = plsc.load_gather(rbuf, [brows[bc], c_vec])
                        o_v[buf, d, t, pl.ds(bc * _LANES, _LANES)] = v

        def start_out(l, buf):
            pltpu.async_copy(o_v.at[buf], out_hbm.at[l, :, wid], semo[buf])

        def wait_out(l, buf):
            pltpu.make_async_copy(o_v.at[buf], out_hbm.at[l, :, wid],
                                  semo[buf]).wait()

        start_gather(0, 0)

        def seq_body(i2, _):
            for phase in range(2):
                l = i2 * 2 + phase
                wait_gather(phase)

                @pl.when(l + 1 < seq)
                def _():
                    start_gather(l + 1, 1 - phase)

                @pl.when(i2 > 0)
                def _():
                    wait_out(l - 2, phase)

                interleave(phase)
                start_out(l, phase)
            return 0

        lax.fori_loop(0, seq // 2, seq_body, 0)
        wait_out(seq - 2, 0)
        wait_out(seq - 1, 1)

    return gather_kernel


def kernel(x, scalar, vector_i, vector_j, vector_k):
    b, l = x.shape
    vocab, dim = scalar.shape
    x_flat = x.reshape(b * l).astype(jnp.int32)
    tables = (scalar, vector_i, vector_j, vector_k)
    tabs_t = [jnp.swapaxes(t, 0, 1) for t in tables]
    n_rem = vocab % _VC
    if n_rem:
        rems = [t[vocab - n_rem:, :].reshape(-1) for t in tables]
    else:
        rems = [jnp.zeros((8 * dim,), jnp.float32)] * 4
    tab = _build_retile(vocab, dim)(*tabs_t, *rems)
    o5 = _build_gather(b, l, vocab, dim)(x_flat, tab)
    return o5.transpose(2, 4, 0, 1, 3).reshape(b, l, dim, 4)
